# baseline (speedup 1.0000x reference)
import jax
import jax.numpy as jnp
from jax.experimental import pallas as pl
from jax.experimental.pallas import tpu as pltpu

_B_BLK = 8


def _body(h, w, x_ref, row_ref, col_ref, out_ref):
    d = col_ref.shape[1]
    col = col_ref[0:w, :]
    row = row_ref[0:h, :]
    col_t = jnp.transpose(col, (1, 0))
    row_t = jnp.transpose(row, (1, 0))
    pos_col = jnp.broadcast_to(col_t[:, None, :], (d, h, w))
    pos_row = jnp.broadcast_to(row_t[:, :, None], (d, h, w))
    pos = jnp.concatenate([pos_col, pos_row], axis=0)
    pos2 = jnp.reshape(pos, (2 * d, h * w))
    out_ref[...] = x_ref[...] + pos2[None]


def kernel(x, row_embed, col_embed):
    b, c2, h, w = x.shape
    hw = h * w
    x3 = x.reshape(b, c2, hw)
    grid = (b // _B_BLK,)
    out = pl.pallas_call(
        lambda *refs: _body(h, w, *refs),
        grid=grid,
        in_specs=[
            pl.BlockSpec((_B_BLK, c2, hw), lambda g: (g, 0, 0)),
            pl.BlockSpec(row_embed.shape, lambda g: (0, 0)),
            pl.BlockSpec(col_embed.shape, lambda g: (0, 0)),
        ],
        out_specs=pl.BlockSpec((_B_BLK, c2, hw), lambda g: (g, 0, 0)),
        out_shape=jax.ShapeDtypeStruct((b, c2, hw), x.dtype),
    )(x3, row_embed, col_embed)
    return out.reshape(b, c2, h, w)


# speedup vs baseline: 1.0004x; 1.0004x over previous
import jax
import jax.numpy as jnp
from jax.experimental import pallas as pl
from jax.experimental.pallas import tpu as pltpu

_B_BLK = 8


def _body(h, w, x_ref, row_ref, col_ref, out_ref, pos_ref):
    d = col_ref.shape[1]

    @pl.when(pl.program_id(0) == 0)
    def _build_pos():
        col = col_ref[0:w, :]
        row = row_ref[0:h, :]
        col_t = jnp.transpose(col, (1, 0))
        row_t = jnp.transpose(row, (1, 0))
        pos_col = jnp.broadcast_to(col_t[:, None, :], (d, h, w))
        pos_row = jnp.broadcast_to(row_t[:, :, None], (d, h, w))
        pos = jnp.concatenate([pos_col, pos_row], axis=0)
        pos_ref[...] = jnp.reshape(pos, (2 * d, h * w))

    out_ref[...] = x_ref[...] + pos_ref[...][None]


def kernel(x, row_embed, col_embed):
    b, c2, h, w = x.shape
    hw = h * w
    x3 = x.reshape(b, c2, hw)
    grid = (b // _B_BLK,)
    out = pl.pallas_call(
        lambda *refs: _body(h, w, *refs),
        grid=grid,
        in_specs=[
            pl.BlockSpec((_B_BLK, c2, hw), lambda g: (g, 0, 0)),
            pl.BlockSpec(row_embed.shape, lambda g: (0, 0)),
            pl.BlockSpec(col_embed.shape, lambda g: (0, 0)),
        ],
        out_specs=pl.BlockSpec((_B_BLK, c2, hw), lambda g: (g, 0, 0)),
        out_shape=jax.ShapeDtypeStruct((b, c2, hw), x.dtype),
        scratch_shapes=[pltpu.VMEM((c2, hw), x.dtype)],
    )(x3, row_embed, col_embed)
    return out.reshape(b, c2, h, w)
